# baseline (speedup 1.0000x reference)
import functools

import jax
import jax.numpy as jnp
from jax import lax
from jax.experimental import pallas as pl
from jax.experimental.pallas import tpu as pltpu
from jax.experimental.pallas import tpu_sc as plsc

_NC = 2
_NS = 16
_NW = _NC * _NS
_DG = 4
_BG = _NW // _DG


def _sc_gather_t(indexes, tableT):
    D, _ = tableT.shape
    B = indexes.shape[0]
    d_per_w = D // _DG
    b_per_w = B // _BG
    mesh = plsc.VectorSubcoreMesh(core_axis_name="c", subcore_axis_name="s")

    @functools.partial(
        pl.kernel,
        out_type=jax.ShapeDtypeStruct((D, B), jnp.float32),
        mesh=mesh,
        compiler_params=pltpu.CompilerParams(use_tc_tiling_on_sc=False),
        scratch_types=[
            pltpu.VMEM((b_per_w,), jnp.int32),
            pltpu.VMEM((d_per_w, b_per_w), jnp.float32),
            pltpu.SemaphoreType.DMA,
            pltpu.SemaphoreType.DMA,
        ],
    )
    def gk(idx_hbm, tab_hbm, out_hbm, idx_v, rows_v, sem_i, sem_g):
        wid = lax.axis_index("s") * _NC + lax.axis_index("c")
        dg = wid // _BG
        bg = wid % _BG
        dbase = dg * d_per_w
        bbase = bg * b_per_w
        pltpu.sync_copy(idx_hbm.at[pl.ds(bbase, b_per_w)], idx_v)
        copies = []
        for k in range(d_per_w):
            copies.append(pltpu.async_copy(
                tab_hbm.at[dbase + k].at[idx_v], rows_v.at[k], sem_g))
        for c in copies:
            c.wait()
        pltpu.sync_copy(
            rows_v, out_hbm.at[pl.ds(dbase, d_per_w), pl.ds(bbase, b_per_w)])

    return gk(indexes, tableT)


def _tc_project_t(embT, W0, W1):
    D, B = embT.shape
    S = W0.shape[0]
    blk = 2048

    def body(x_ref, w0_ref, w1_ref, o_ref):
        x = x_ref[...]
        dn = (((1,), (0,)), ((), ()))
        o_ref[0] = lax.dot_general(
            w0_ref[...], x, dn, preferred_element_type=jnp.float32)
        o_ref[1] = lax.dot_general(
            w1_ref[...], x, dn, preferred_element_type=jnp.float32)

    return pl.pallas_call(
        body,
        grid=(B // blk,),
        in_specs=[
            pl.BlockSpec((D, blk), lambda i: (0, i)),
            pl.BlockSpec((S, D), lambda i: (0, 0)),
            pl.BlockSpec((S, D), lambda i: (0, 0)),
        ],
        out_specs=pl.BlockSpec((2, S, blk), lambda i: (0, 0, i)),
        out_shape=jax.ShapeDtypeStruct((2, S, B), jnp.float32),
    )(embT, W0, W1)


def kernel(indexes, table, W0, W1):
    indexes = indexes.astype(jnp.int32)
    tableT = jnp.transpose(table)
    embT = _sc_gather_t(indexes, tableT)
    outT = _tc_project_t(embT, W0, W1)
    return jnp.transpose(outT, (0, 2, 1))


# speedup vs baseline: 19.2475x; 19.2475x over previous
import functools

import jax
import jax.numpy as jnp
from jax import lax
from jax.experimental import pallas as pl
from jax.experimental.pallas import tpu as pltpu
from jax.experimental.pallas import tpu_sc as plsc
from jax.experimental.layout import Format, Layout, with_layout_constraint

_NC = 2
_NS = 16
_NW = _NC * _NS
_DG = 4
_BG = _NW // _DG


def _sc_gather_t(indexes, tableT):
    D, _ = tableT.shape
    B = indexes.shape[0]
    d_per_w = D // _DG
    b_per_w = B // _BG
    mesh = plsc.VectorSubcoreMesh(core_axis_name="c", subcore_axis_name="s")

    @functools.partial(
        pl.kernel,
        out_type=jax.ShapeDtypeStruct((D, B), jnp.float32),
        mesh=mesh,
        compiler_params=pltpu.CompilerParams(use_tc_tiling_on_sc=False),
        scratch_types=[
            pltpu.VMEM((b_per_w,), jnp.int32),
            pltpu.VMEM((d_per_w, b_per_w), jnp.float32),
            pltpu.SemaphoreType.DMA,
            pltpu.SemaphoreType.DMA,
        ],
    )
    def gk(idx_hbm, tab_hbm, out_hbm, idx_v, rows_v, sem_i, sem_g):
        wid = lax.axis_index("s") * _NC + lax.axis_index("c")
        dg = wid // _BG
        bg = wid % _BG
        dbase = dg * d_per_w
        bbase = bg * b_per_w
        pltpu.sync_copy(idx_hbm.at[pl.ds(bbase, b_per_w)], idx_v)
        copies = []
        for k in range(d_per_w):
            copies.append(pltpu.async_copy(
                tab_hbm.at[dbase + k].at[idx_v], rows_v.at[k], sem_g))
        for c in copies:
            c.wait()
        pltpu.sync_copy(
            rows_v, out_hbm.at[pl.ds(dbase, d_per_w), pl.ds(bbase, b_per_w)])

    return gk(indexes, tableT)


def _tc_project_t(embT, W0, W1):
    D, B = embT.shape
    S = W0.shape[0]
    blk = 2048

    def body(x_ref, w0_ref, w1_ref, o_ref):
        x = x_ref[...]
        dn = (((1,), (0,)), ((), ()))
        o_ref[0] = lax.dot_general(
            w0_ref[...], x, dn, preferred_element_type=jnp.float32)
        o_ref[1] = lax.dot_general(
            w1_ref[...], x, dn, preferred_element_type=jnp.float32)

    return pl.pallas_call(
        body,
        grid=(B // blk,),
        in_specs=[
            pl.BlockSpec((D, blk), lambda i: (0, i)),
            pl.BlockSpec((S, D), lambda i: (0, 0)),
            pl.BlockSpec((S, D), lambda i: (0, 0)),
        ],
        out_specs=pl.BlockSpec((2, S, blk), lambda i: (0, 0, i)),
        out_shape=jax.ShapeDtypeStruct((2, S, B), jnp.float32),
    )(embT, W0, W1)


def kernel(indexes, table, W0, W1):
    indexes = indexes.astype(jnp.int32)
    tableT = with_layout_constraint(
        jnp.transpose(table),
        Layout(major_to_minor=(0, 1), tiling=((8, 128),)))
    embT = _sc_gather_t(indexes, tableT)
    outT = _tc_project_t(embT, W0, W1)
    return jnp.transpose(outT, (0, 2, 1))
